# Initial kernel scaffold; baseline (speedup 1.0000x reference)
import jax
import jax.numpy as jnp
from jax.experimental import pallas as pl


def kernel(node_pos, full_edge_index, full_edge_attr, batch):
    raise NotImplementedError("write your pallas kernel here")


# speedup vs baseline: 145.5383x; 145.5383x over previous
import functools

import jax
import jax.numpy as jnp
from jax import lax
from jax.experimental import pallas as pl
from jax.experimental.pallas import tpu as pltpu
from jax.experimental.pallas import tpu_sc as plsc

NN = 100000
NE = 6400000
NG = 128
LANES = 16
NW = 32
EPT = NE // NW
C = 160
VPC = C // LANES
NCHUNK = EPT // C
BWORDS = NN // 4
BPAD = BWORDS + 8


def _rsqrt_nr(v):
    bits = plsc.bitcast(v, jnp.int32)
    r = plsc.bitcast(jnp.int32(0x5F3759DF) - (bits >> 1), jnp.float32)
    h = v * jnp.float32(0.5)
    r = r * (jnp.float32(1.5) - h * r * r)
    r = r * (jnp.float32(1.5) - h * r * r)
    return r


def _recip_nr(v):
    bits = plsc.bitcast(v, jnp.int32)
    x = plsc.bitcast(jnp.int32(0x7EF127EA) - bits, jnp.float32)
    x = x * (jnp.float32(2.0) - v * x)
    x = x * (jnp.float32(2.0) - v * x)
    return x


def _decode_pos(w):
    x = plsc.bitcast(lax.shift_left(w, 16), jnp.float32)
    y = plsc.bitcast(lax.shift_left(lax.shift_right_arithmetic(w, 16), 16),
                     jnp.float32)
    return x, y


def _graph_of(batch_tbl, nv):
    word = plsc.load_gather(batch_tbl, [lax.shift_right_arithmetic(nv, 2)])
    sh = lax.shift_left(nv & 3, 3)
    return lax.shift_right_logical(word, sh) & 127


def _zero_acc(acc):
    z = jnp.zeros((LANES,), jnp.float32)
    for g in range(NG):
        acc[pl.ds(g * LANES, LANES)] = z


def _wid():
    return lax.axis_index("s") * 2 + lax.axis_index("c")


def _edge_loop(fei_hbm, d_hbm, bsrc, bdst, bd, sems, per_vector):
    base = _wid() * EPT

    def issue(slot, chunk):
        off = base + chunk * C
        pltpu.async_copy(fei_hbm.at[pl.ds(off, C)], bsrc[slot], sems[slot])
        pltpu.async_copy(fei_hbm.at[pl.ds(NE + off, C)], bdst[slot], sems[slot])
        pltpu.async_copy(d_hbm.at[pl.ds(off, C)], bd[slot], sems[slot])

    def drain(slot):
        pltpu.make_async_copy(fei_hbm.at[pl.ds(0, C)], bsrc[slot], sems[slot]).wait()
        pltpu.make_async_copy(fei_hbm.at[pl.ds(0, C)], bdst[slot], sems[slot]).wait()
        pltpu.make_async_copy(d_hbm.at[pl.ds(0, C)], bd[slot], sems[slot]).wait()

    issue(0, 0)
    issue(1, 1)

    def body(j, carry):
        for slot in range(2):
            chunk = j * 2 + slot
            drain(slot)
            for k in range(VPC):
                sv = bsrc[slot][pl.ds(k * LANES, LANES)]
                dv = bdst[slot][pl.ds(k * LANES, LANES)]
                dval = bd[slot][pl.ds(k * LANES, LANES)]
                per_vector(sv, dv, dval)

            @pl.when(chunk + 2 < NCHUNK)
            def _():
                issue(slot, chunk + 2)
        return carry

    lax.fori_loop(0, NCHUNK // 2, body, 0)


@functools.partial(
    pl.kernel,
    mesh=plsc.VectorSubcoreMesh(core_axis_name="c", subcore_axis_name="s"),
    compiler_params=pltpu.CompilerParams(needs_layout_passes=False),
    out_type=[
        jax.ShapeDtypeStruct((NW, NG * LANES), jnp.float32),
        jax.ShapeDtypeStruct((NW, NG * LANES), jnp.float32),
        jax.ShapeDtypeStruct((NW, NG * LANES), jnp.float32),
    ],
    scratch_types=[
        pltpu.VMEM((NN,), jnp.int32),
        pltpu.VMEM((BPAD,), jnp.int32),
        pltpu.VMEM((C,), jnp.int32),
        pltpu.VMEM((C,), jnp.int32),
        pltpu.VMEM((C,), jnp.int32),
        pltpu.VMEM((C,), jnp.int32),
        pltpu.VMEM((C,), jnp.float32),
        pltpu.VMEM((C,), jnp.float32),
        pltpu.VMEM((NG * LANES,), jnp.float32),
        pltpu.VMEM((NG * LANES,), jnp.float32),
        pltpu.SemaphoreType.DMA,
        pltpu.SemaphoreType.DMA,
    ],
)
def _pass1(pos_w_hbm, batch_w_hbm, fei_hbm, d_hbm,
           num_out, den_out, sz_out,
           pos_tbl, batch_tbl, bsrc0, bsrc1, bdst0, bdst1, bd0, bd1,
           acc_num, acc_den, sem0, sem1):
    bsrc, bdst, bd = (bsrc0, bsrc1), (bdst0, bdst1), (bd0, bd1)
    wid = _wid()
    pltpu.sync_copy(batch_w_hbm, batch_tbl)
    pltpu.sync_copy(pos_w_hbm, pos_tbl)
    _zero_acc(acc_num)
    _zero_acc(acc_den)
    lane = lax.iota(jnp.int32, LANES)
    ones = jnp.ones((LANES,), jnp.float32)

    vstart = 49 * wid - jnp.maximum(wid - 27, 0)
    vcount = 49 - (wid >= 27).astype(jnp.int32)

    def szbody(v, carry):
        w = batch_tbl[pl.ds(v * LANES, LANES)]
        for b in range(4):
            g = lax.shift_right_logical(w, 8 * b) & 127
            plsc.addupdate_scatter(acc_num, [lax.shift_left(g, 4) | lane], ones)
        return carry

    lax.fori_loop(vstart, vstart + vcount, szbody, 0)
    pltpu.sync_copy(acc_num, sz_out.at[wid])
    _zero_acc(acc_num)

    def per_vector(sv, dv, dval):
        ws = plsc.load_gather(pos_tbl, [sv])
        wd = plsc.load_gather(pos_tbl, [dv])
        xs, ys = _decode_pos(ws)
        xd, yd = _decode_pos(wd)
        gs = _graph_of(batch_tbl, sv)
        dx = xs - xd
        dy = ys - yd
        e2 = dx * dx + dy * dy
        eu = e2 * _rsqrt_nr(e2)
        ratio = eu * _recip_nr(dval)
        slot = lax.shift_left(gs, 4) | lane
        plsc.addupdate_scatter(acc_num, [slot], ratio * ratio)
        plsc.addupdate_scatter(acc_den, [slot], ratio)

    _edge_loop(fei_hbm, d_hbm, bsrc, bdst, bd, (sem0, sem1), per_vector)
    pltpu.sync_copy(acc_num, num_out.at[wid])
    pltpu.sync_copy(acc_den, den_out.at[wid])


@functools.partial(
    pl.kernel,
    mesh=plsc.VectorSubcoreMesh(core_axis_name="c", subcore_axis_name="s"),
    compiler_params=pltpu.CompilerParams(needs_layout_passes=False),
    out_type=jax.ShapeDtypeStruct((NW, NG * LANES), jnp.float32),
    scratch_types=[
        pltpu.VMEM((NN,), jnp.int32),
        pltpu.VMEM((BPAD,), jnp.int32),
        pltpu.VMEM((NG,), jnp.float32),
        pltpu.VMEM((C,), jnp.int32),
        pltpu.VMEM((C,), jnp.int32),
        pltpu.VMEM((C,), jnp.int32),
        pltpu.VMEM((C,), jnp.int32),
        pltpu.VMEM((C,), jnp.float32),
        pltpu.VMEM((C,), jnp.float32),
        pltpu.VMEM((NG * LANES,), jnp.float32),
        pltpu.SemaphoreType.DMA,
        pltpu.SemaphoreType.DMA,
    ],
)
def _pass2(pos_w_hbm, batch_w_hbm, fei_hbm, d_hbm, invs_hbm,
           st_out,
           pos_tbl, batch_tbl, invs_tbl, bsrc0, bsrc1, bdst0, bdst1, bd0, bd1,
           acc, sem0, sem1):
    bsrc, bdst, bd = (bsrc0, bsrc1), (bdst0, bdst1), (bd0, bd1)
    wid = _wid()
    pltpu.sync_copy(batch_w_hbm, batch_tbl)
    pltpu.sync_copy(pos_w_hbm, pos_tbl)
    pltpu.sync_copy(invs_hbm, invs_tbl)
    _zero_acc(acc)
    lane = lax.iota(jnp.int32, LANES)

    def per_vector(sv, dv, dval):
        ws = plsc.load_gather(pos_tbl, [sv])
        wd = plsc.load_gather(pos_tbl, [dv])
        xs, ys = _decode_pos(ws)
        xd, yd = _decode_pos(wd)
        gs = _graph_of(batch_tbl, sv)
        gd = _graph_of(batch_tbl, dv)
        a = plsc.load_gather(invs_tbl, [gs])
        b = plsc.load_gather(invs_tbl, [gd])
        dx = xs * a - xd * b
        dy = ys * a - yd * b
        e2 = dx * dx + dy * dy
        eu = e2 * _rsqrt_nr(e2)
        t = eu * _recip_nr(dval) - jnp.float32(1.0)
        plsc.addupdate_scatter(acc, [lax.shift_left(gs, 4) | lane], t * t)

    _edge_loop(fei_hbm, d_hbm, bsrc, bdst, bd, (sem0, sem1), per_vector)
    pltpu.sync_copy(acc, st_out.at[wid])


def kernel(node_pos, full_edge_index, full_edge_attr, batch):
    xb = lax.bitcast_convert_type(node_pos[:, 0].astype(jnp.bfloat16), jnp.uint16)
    yb = lax.bitcast_convert_type(node_pos[:, 1].astype(jnp.bfloat16), jnp.uint16)
    pos_w = lax.bitcast_convert_type(
        xb.astype(jnp.uint32) | (yb.astype(jnp.uint32) << 16), jnp.int32)
    bu = batch.astype(jnp.uint32).reshape(BWORDS, 4)
    bw = bu[:, 0] | (bu[:, 1] << 8) | (bu[:, 2] << 16) | (bu[:, 3] << 24)
    batch_w = jnp.concatenate(
        [lax.bitcast_convert_type(bw, jnp.int32),
         jnp.full((8,), 0x7F7F7F7F, jnp.int32)])
    fei_flat = full_edge_index.reshape(-1)
    dcol = full_edge_attr[:, 0]

    num_p, den_p, sz_p = _pass1(pos_w, batch_w, fei_flat, dcol)
    num = jnp.sum(num_p.reshape(NW, NG, LANES), axis=(0, 2))
    den = jnp.sum(den_p.reshape(NW, NG, LANES), axis=(0, 2))
    sizes = jnp.sum(sz_p.reshape(NW, NG, LANES), axis=(0, 2))
    sizes = sizes.at[NG - 1].add(-32.0)
    invs = den / num

    st_p = _pass2(pos_w, batch_w, fei_flat, dcol, invs)
    graph_stress = jnp.sum(st_p.reshape(NW, NG, LANES), axis=(0, 2)) / (sizes * sizes)
    return jnp.mean(graph_stress)
